# baseline (speedup 1.0000x reference)
import functools

import numpy as _np
import jax
import jax.numpy as jnp
from jax import lax
from jax.experimental import pallas as pl
from jax.experimental.pallas import tpu as pltpu
from jax.experimental.pallas import tpu_sc as plsc

N = 10000
NPAD = 10240
E = 320000
NC = 2
NS = 16
NW = NC * NS
CHUNK = 128
RPT = 80
EPAD = NW * RPT * CHUNK
ROWS = NW * RPT
PER_SC = NPAD // NS

_mesh = plsc.VectorSubcoreMesh(core_axis_name="c", subcore_axis_name="s")


def _wid(cid, sid):
    return sid * NC + cid


@functools.partial(
    pl.kernel,
    out_type=jax.ShapeDtypeStruct((NC * 8, NPAD), jnp.float32),
    mesh=_mesh,
    scratch_types=[
        pltpu.VMEM((RPT, CHUNK), jnp.int32),
        pltpu.VMEM((RPT, CHUNK), jnp.float32),
        pltpu.VMEM((PER_SC,), jnp.float32),
        pltpu.VMEM_SHARED((NPAD,), jnp.float32),
    ],
)
def _deg_kernel(dst_hbm, ew_hbm, out_hbm, dst_v, ew_v, zbuf, acc):
    cid = lax.axis_index("c")
    sid = lax.axis_index("s")
    wid = _wid(cid, sid)

    zeros16 = jnp.zeros((16,), jnp.float32)
    for g in range(PER_SC // 16):
        zbuf[pl.ds(g * 16, 16)] = zeros16
    pltpu.sync_copy(zbuf, acc.at[pl.ds(sid * PER_SC, PER_SC)])
    plsc.subcore_barrier()

    pltpu.sync_copy(dst_hbm.at[pl.ds(wid * RPT, RPT)], dst_v)
    pltpu.sync_copy(ew_hbm.at[pl.ds(wid * RPT, RPT)], ew_v)

    def body(j, carry):
        pltpu.sync_copy(ew_v.at[j], acc.at[dst_v.at[j]], add=True)
        return carry

    lax.fori_loop(0, RPT, body, 0)
    plsc.subcore_barrier()

    pltpu.sync_copy(acc.at[pl.ds(sid * PER_SC, PER_SC)],
                    out_hbm.at[cid * 8, pl.ds(sid * PER_SC, PER_SC)])


def _make_agg_kernel(D):
    epg = 16 // D
    ngroups = (CHUNK * D) // 16
    logd = D.bit_length() - 1

    @functools.partial(
        pl.kernel,
        out_type=jax.ShapeDtypeStruct((NC, NPAD, D), jnp.float32),
        mesh=_mesh,
        compiler_params=pltpu.CompilerParams(
            use_tc_tiling_on_sc=False, needs_layout_passes=False),
        scratch_types=[
            pltpu.VMEM((RPT, CHUNK), jnp.int32),
            pltpu.VMEM((RPT, CHUNK), jnp.int32),
            pltpu.VMEM((RPT, CHUNK), jnp.float32),
            pltpu.VMEM((CHUNK, D), jnp.float32),
            pltpu.VMEM((CHUNK, D), jnp.float32),
            pltpu.VMEM((CHUNK, D), jnp.float32),
            pltpu.VMEM((CHUNK, D), jnp.float32),
            pltpu.VMEM((CHUNK, D), jnp.float32),
            pltpu.VMEM_SHARED((NPAD, D), jnp.float32),
            pltpu.SemaphoreType.DMA,
            pltpu.SemaphoreType.DMA,
            pltpu.SemaphoreType.DMA,
            pltpu.SemaphoreType.DMA,
        ],
    )
    def _agg(src_hbm, dst_hbm, ew_hbm, h_hbm, out_hbm,
             src_v, dst_v, ew_v, rows0, rows1, rows2, rows3, msg_v, acc,
             sem0, sem1, sem2, sem3):
        cid = lax.axis_index("c")
        sid = lax.axis_index("s")
        wid = _wid(cid, sid)

        zeros16 = jnp.zeros((16,), jnp.float32)
        iota = lax.iota(jnp.int32, 16)
        erel = lax.shift_right_logical(iota, logd)
        cidx_c = lax.bitwise_and(iota, D - 1)
        eidx_c = [erel + g * epg for g in range(ngroups)]

        for g in range(ngroups):
            plsc.store_scatter(msg_v, [eidx_c[g], cidx_c], zeros16)
        for r in range(PER_SC // CHUNK):
            pltpu.sync_copy(
                msg_v, acc.at[pl.ds(sid * PER_SC + r * CHUNK, CHUNK)])
        plsc.subcore_barrier()

        pltpu.sync_copy(src_hbm.at[pl.ds(wid * RPT, RPT)], src_v)
        pltpu.sync_copy(dst_hbm.at[pl.ds(wid * RPT, RPT)], dst_v)
        pltpu.sync_copy(ew_hbm.at[pl.ds(wid * RPT, RPT)], ew_v)

        rows = (rows0, rows1, rows2, rows3)
        sems = (sem0, sem1, sem2, sem3)
        NBUF = 4

        for b in range(NBUF):
            pltpu.async_copy(h_hbm.at[src_v.at[b]], rows[b], sems[b])

        def _process(j, rv):
            j16 = jnp.full((16,), j, jnp.int32)
            for g in range(ngroups):
                ewb = plsc.load_gather(ew_v, [j16, eidx_c[g]])
                vals = plsc.load_gather(rv, [eidx_c[g], cidx_c])
                plsc.store_scatter(msg_v, [eidx_c[g], cidx_c], ewb * vals)
            pltpu.sync_copy(msg_v, acc.at[dst_v.at[j]], add=True)

        def body(i, carry):
            j0 = NBUF * i
            for b in range(NBUF):
                j = j0 + b
                pltpu.make_async_copy(
                    h_hbm.at[src_v.at[j]], rows[b], sems[b]).wait()
                _process(j, rows[b])
                jn = jnp.minimum(j + NBUF, RPT - 1)

                @pl.when(j + NBUF < RPT)
                def _():
                    pltpu.async_copy(h_hbm.at[src_v.at[jn]], rows[b], sems[b])

            return carry

        lax.fori_loop(0, RPT // NBUF, body, 0)
        plsc.subcore_barrier()

        pltpu.sync_copy(acc.at[pl.ds(sid * PER_SC, PER_SC)],
                        out_hbm.at[cid, pl.ds(sid * PER_SC, PER_SC)])

    return _agg


_agg16 = _make_agg_kernel(16)


_BLK = 1000
_GRID = N // _BLK


def _tc_a_body(degp_ref, x_ref, w1_ref, dinv_ref, h1s_ref):
    deg = 1.0 + degp_ref[0] + degp_ref[1]
    dinv = lax.rsqrt(deg)
    dinv_ref[...] = dinv
    h = jnp.dot(x_ref[...], w1_ref[...], preferred_element_type=jnp.float32)
    h1s_ref[...] = h * dinv


def _tc_b_body(s1p_ref, h1s_ref, dinv_ref, b1_ref, w2p_ref, h2s_ref):
    dinv = dinv_ref[...]
    pre = dinv * (s1p_ref[0] + s1p_ref[1] + h1s_ref[...]) + b1_ref[...]
    z = jnp.maximum(pre, 0.0)
    h2 = jnp.dot(z, w2p_ref[...], preferred_element_type=jnp.float32)
    h2s_ref[...] = h2 * dinv


def _tc_c_body(s2p_ref, h2s_ref, dinv_ref, b2_ref, out_ref):
    o2 = dinv_ref[...] * (s2p_ref[0] + s2p_ref[1] + h2s_ref[...]) + b2_ref[...]
    m = jnp.max(o2, axis=1, keepdims=True)
    lse = m + jnp.log(jnp.sum(jnp.exp(o2 - m), axis=1, keepdims=True))
    out_ref[...] = o2 - lse


def kernel(x, edge_index, edge_weight, W1, b1, W2, b2):
    f32 = jnp.float32
    src = edge_index[0].astype(jnp.int32)
    dst = edge_index[1].astype(jnp.int32)
    ew = edge_weight.astype(f32)

    pad = EPAD - E
    pad_idx = jnp.arange(pad, dtype=jnp.int32) % N
    src2d = jnp.concatenate([src, pad_idx]).reshape(ROWS, CHUNK)
    dst2d = jnp.concatenate([dst, pad_idx]).reshape(ROWS, CHUNK)
    ew2d = jnp.concatenate([ew, jnp.zeros((pad,), f32)]).reshape(ROWS, CHUNK)

    w2p = jnp.concatenate([W2, jnp.zeros((16, 14), f32)], axis=1)

    degp = _deg_kernel(dst2d, ew2d)[::8]
    degp3 = degp[:, :N, None]

    dinv, h1s = pl.pallas_call(
        _tc_a_body,
        grid=(_GRID,),
        in_specs=[
            pl.BlockSpec((NC, _BLK, 1), lambda i: (0, i, 0)),
            pl.BlockSpec((_BLK, 128), lambda i: (i, 0)),
            pl.BlockSpec((128, 16), lambda i: (0, 0)),
        ],
        out_specs=[
            pl.BlockSpec((_BLK, 1), lambda i: (i, 0)),
            pl.BlockSpec((_BLK, 16), lambda i: (i, 0)),
        ],
        out_shape=[
            jax.ShapeDtypeStruct((N, 1), f32),
            jax.ShapeDtypeStruct((N, 16), f32),
        ],
    )(degp3, x, W1)

    s1p = _agg16(src2d, dst2d, ew2d, h1s)
    s1p = s1p[:, :N, :]

    h2s = pl.pallas_call(
        _tc_b_body,
        grid=(_GRID,),
        in_specs=[
            pl.BlockSpec((NC, _BLK, 16), lambda i: (0, i, 0)),
            pl.BlockSpec((_BLK, 16), lambda i: (i, 0)),
            pl.BlockSpec((_BLK, 1), lambda i: (i, 0)),
            pl.BlockSpec((16,), lambda i: (0,)),
            pl.BlockSpec((16, 16), lambda i: (0, 0)),
        ],
        out_specs=pl.BlockSpec((_BLK, 16), lambda i: (i, 0)),
        out_shape=jax.ShapeDtypeStruct((N, 16), f32),
    )(s1p, h1s, dinv, b1, w2p)

    s2p = _agg16(src2d, dst2d, ew2d, h2s)
    s2p = s2p[:, :N, :2]
    h2s2 = h2s[:, :2]

    out = pl.pallas_call(
        _tc_c_body,
        grid=(_GRID,),
        in_specs=[
            pl.BlockSpec((NC, _BLK, 2), lambda i: (0, i, 0)),
            pl.BlockSpec((_BLK, 2), lambda i: (i, 0)),
            pl.BlockSpec((_BLK, 1), lambda i: (i, 0)),
            pl.BlockSpec((2,), lambda i: (0,)),
        ],
        out_specs=pl.BlockSpec((_BLK, 2), lambda i: (i, 0)),
        out_shape=jax.ShapeDtypeStruct((N, 2), f32),
    )(s2p, h2s2, dinv, b2)

    return out


# speedup vs baseline: 1.4062x; 1.4062x over previous
import functools

import numpy as _np
import jax
import jax.numpy as jnp
from jax import lax
from jax.experimental import pallas as pl
from jax.experimental.pallas import tpu as pltpu
from jax.experimental.pallas import tpu_sc as plsc

N = 10000
NPAD = 10240
E = 320000
NC = 2
NS = 16
NW = NC * NS
CHUNK = 128
RPT = 80
EPAD = NW * RPT * CHUNK
ROWS = NW * RPT
PER_SC = NPAD // NS

_mesh = plsc.VectorSubcoreMesh(core_axis_name="c", subcore_axis_name="s")


def _wid(cid, sid):
    return sid * NC + cid


@functools.partial(
    pl.kernel,
    out_type=jax.ShapeDtypeStruct((NC * 8, NPAD), jnp.float32),
    mesh=_mesh,
    scratch_types=[
        pltpu.VMEM((RPT, CHUNK), jnp.int32),
        pltpu.VMEM((RPT, CHUNK), jnp.float32),
        pltpu.VMEM((PER_SC,), jnp.float32),
        pltpu.VMEM_SHARED((NPAD,), jnp.float32),
    ],
)
def _deg_kernel(dst_hbm, ew_hbm, out_hbm, dst_v, ew_v, zbuf, acc):
    cid = lax.axis_index("c")
    sid = lax.axis_index("s")
    wid = _wid(cid, sid)

    zeros16 = jnp.zeros((16,), jnp.float32)
    for g in range(PER_SC // 16):
        zbuf[pl.ds(g * 16, 16)] = zeros16
    pltpu.sync_copy(zbuf, acc.at[pl.ds(sid * PER_SC, PER_SC)])
    plsc.subcore_barrier()

    pltpu.sync_copy(dst_hbm.at[pl.ds(wid * RPT, RPT)], dst_v)
    pltpu.sync_copy(ew_hbm.at[pl.ds(wid * RPT, RPT)], ew_v)

    def body(j, carry):
        pltpu.sync_copy(ew_v.at[j], acc.at[dst_v.at[j]], add=True)
        return carry

    lax.fori_loop(0, RPT, body, 0)
    plsc.subcore_barrier()

    pltpu.sync_copy(acc.at[pl.ds(sid * PER_SC, PER_SC)],
                    out_hbm.at[cid * 8, pl.ds(sid * PER_SC, PER_SC)])


def _make_agg_kernel(D):
    epg = 16 // D
    ngroups = (CHUNK * D) // 16
    logd = D.bit_length() - 1

    @functools.partial(
        pl.kernel,
        out_type=jax.ShapeDtypeStruct((NC, NPAD, D), jnp.float32),
        mesh=_mesh,
        compiler_params=pltpu.CompilerParams(
            use_tc_tiling_on_sc=False, needs_layout_passes=False),
        scratch_types=[
            pltpu.VMEM((RPT, CHUNK), jnp.int32),
            pltpu.VMEM((RPT, CHUNK), jnp.int32),
            pltpu.VMEM((RPT, CHUNK), jnp.float32),
            pltpu.VMEM((CHUNK, D), jnp.float32),
            pltpu.VMEM((CHUNK, D), jnp.float32),
            pltpu.VMEM((CHUNK, D), jnp.float32),
            pltpu.VMEM_SHARED((NPAD, D), jnp.float32),
            pltpu.SemaphoreType.DMA,
            pltpu.SemaphoreType.DMA,
        ],
    )
    def _agg(src_hbm, dst_hbm, ew_hbm, h_hbm, out_hbm,
             src_v, dst_v, ew_v, rows0, rows1, msg_v, acc, sem0, sem1):
        cid = lax.axis_index("c")
        sid = lax.axis_index("s")
        wid = _wid(cid, sid)

        zeros16 = jnp.zeros((16,), jnp.float32)
        iota = lax.iota(jnp.int32, 16)
        erel = lax.shift_right_logical(iota, logd)
        cidx_c = lax.bitwise_and(iota, D - 1)
        eidx_c = [erel + g * epg for g in range(ngroups)]

        for g in range(ngroups):
            plsc.store_scatter(msg_v, [eidx_c[g], cidx_c], zeros16)
        for r in range(PER_SC // CHUNK):
            pltpu.sync_copy(
                msg_v, acc.at[pl.ds(sid * PER_SC + r * CHUNK, CHUNK)])
        plsc.subcore_barrier()

        pltpu.sync_copy(src_hbm.at[pl.ds(wid * RPT, RPT)], src_v)
        pltpu.sync_copy(dst_hbm.at[pl.ds(wid * RPT, RPT)], dst_v)
        pltpu.sync_copy(ew_hbm.at[pl.ds(wid * RPT, RPT)], ew_v)

        rows = (rows0, rows1)
        sems = (sem0, sem1)
        NBUF = 2

        for b in range(NBUF):
            pltpu.async_copy(h_hbm.at[src_v.at[b]], rows[b], sems[b])

        def _process(j, rv):
            j16 = jnp.full((16,), j, jnp.int32)
            for g in range(ngroups):
                ewb = plsc.load_gather(ew_v, [j16, eidx_c[g]])
                if D == 16:
                    msg_v[g] = ewb * rv[g]
                else:
                    vals = plsc.load_gather(rv, [eidx_c[g], cidx_c])
                    plsc.store_scatter(msg_v, [eidx_c[g], cidx_c], ewb * vals)
            pltpu.sync_copy(msg_v, acc.at[dst_v.at[j]], add=True)

        def body(i, carry):
            j0 = NBUF * i
            for b in range(NBUF):
                j = j0 + b
                pltpu.make_async_copy(
                    h_hbm.at[src_v.at[j]], rows[b], sems[b]).wait()
                _process(j, rows[b])
                jn = jnp.minimum(j + NBUF, RPT - 1)

                @pl.when(j + NBUF < RPT)
                def _():
                    pltpu.async_copy(h_hbm.at[src_v.at[jn]], rows[b], sems[b])

            return carry

        lax.fori_loop(0, RPT // NBUF, body, 0)
        plsc.subcore_barrier()

        pltpu.sync_copy(acc.at[pl.ds(sid * PER_SC, PER_SC)],
                        out_hbm.at[cid, pl.ds(sid * PER_SC, PER_SC)])

    return _agg


_agg16 = _make_agg_kernel(16)


_BLK = 1000
_GRID = N // _BLK


def _tc_a_body(degp_ref, x_ref, w1_ref, dinv_ref, h1s_ref):
    deg = 1.0 + degp_ref[0] + degp_ref[1]
    dinv = lax.rsqrt(deg)
    dinv_ref[...] = dinv
    h = jnp.dot(x_ref[...], w1_ref[...], preferred_element_type=jnp.float32)
    h1s_ref[...] = h * dinv


def _tc_b_body(s1p_ref, h1s_ref, dinv_ref, b1_ref, w2p_ref, h2s_ref):
    dinv = dinv_ref[...]
    pre = dinv * (s1p_ref[0] + s1p_ref[1] + h1s_ref[...]) + b1_ref[...]
    z = jnp.maximum(pre, 0.0)
    h2 = jnp.dot(z, w2p_ref[...], preferred_element_type=jnp.float32)
    h2s_ref[...] = h2 * dinv


def _tc_c_body(s2p_ref, h2s_ref, dinv_ref, b2_ref, out_ref):
    o2 = dinv_ref[...] * (s2p_ref[0] + s2p_ref[1] + h2s_ref[...]) + b2_ref[...]
    m = jnp.max(o2, axis=1, keepdims=True)
    lse = m + jnp.log(jnp.sum(jnp.exp(o2 - m), axis=1, keepdims=True))
    out_ref[...] = o2 - lse


def kernel(x, edge_index, edge_weight, W1, b1, W2, b2):
    f32 = jnp.float32
    src = edge_index[0].astype(jnp.int32)
    dst = edge_index[1].astype(jnp.int32)
    ew = edge_weight.astype(f32)

    pad = EPAD - E
    pad_idx = jnp.arange(pad, dtype=jnp.int32) % N
    src2d = jnp.concatenate([src, pad_idx]).reshape(ROWS, CHUNK)
    dst2d = jnp.concatenate([dst, pad_idx]).reshape(ROWS, CHUNK)
    ew2d = jnp.concatenate([ew, jnp.zeros((pad,), f32)]).reshape(ROWS, CHUNK)

    w2p = jnp.concatenate([W2, jnp.zeros((16, 14), f32)], axis=1)

    degp = _deg_kernel(dst2d, ew2d)[::8]
    degp3 = degp[:, :N, None]

    dinv, h1s = pl.pallas_call(
        _tc_a_body,
        grid=(_GRID,),
        in_specs=[
            pl.BlockSpec((NC, _BLK, 1), lambda i: (0, i, 0)),
            pl.BlockSpec((_BLK, 128), lambda i: (i, 0)),
            pl.BlockSpec((128, 16), lambda i: (0, 0)),
        ],
        out_specs=[
            pl.BlockSpec((_BLK, 1), lambda i: (i, 0)),
            pl.BlockSpec((_BLK, 16), lambda i: (i, 0)),
        ],
        out_shape=[
            jax.ShapeDtypeStruct((N, 1), f32),
            jax.ShapeDtypeStruct((N, 16), f32),
        ],
    )(degp3, x, W1)

    s1p = _agg16(src2d, dst2d, ew2d, h1s)
    s1p = s1p[:, :N, :]

    h2s = pl.pallas_call(
        _tc_b_body,
        grid=(_GRID,),
        in_specs=[
            pl.BlockSpec((NC, _BLK, 16), lambda i: (0, i, 0)),
            pl.BlockSpec((_BLK, 16), lambda i: (i, 0)),
            pl.BlockSpec((_BLK, 1), lambda i: (i, 0)),
            pl.BlockSpec((16,), lambda i: (0,)),
            pl.BlockSpec((16, 16), lambda i: (0, 0)),
        ],
        out_specs=pl.BlockSpec((_BLK, 16), lambda i: (i, 0)),
        out_shape=jax.ShapeDtypeStruct((N, 16), f32),
    )(s1p, h1s, dinv, b1, w2p)

    s2p = _agg16(src2d, dst2d, ew2d, h2s)
    s2p = s2p[:, :N, :2]
    h2s2 = h2s[:, :2]

    out = pl.pallas_call(
        _tc_c_body,
        grid=(_GRID,),
        in_specs=[
            pl.BlockSpec((NC, _BLK, 2), lambda i: (0, i, 0)),
            pl.BlockSpec((_BLK, 2), lambda i: (i, 0)),
            pl.BlockSpec((_BLK, 1), lambda i: (i, 0)),
            pl.BlockSpec((2,), lambda i: (0,)),
        ],
        out_specs=pl.BlockSpec((_BLK, 2), lambda i: (i, 0)),
        out_shape=jax.ShapeDtypeStruct((N, 2), f32),
    )(s2p, h2s2, dinv, b2)

    return out


# speedup vs baseline: 1.4662x; 1.0427x over previous
import functools

import numpy as _np
import jax
import jax.numpy as jnp
from jax import lax
from jax.experimental import pallas as pl
from jax.experimental.pallas import tpu as pltpu
from jax.experimental.pallas import tpu_sc as plsc

N = 10000
NPAD = 10240
E = 320000
NC = 2
NS = 16
NW = NC * NS
CHUNK = 128
RPT = 80
EPAD = NW * RPT * CHUNK
ROWS = NW * RPT
PER_SC = NPAD // NS

_mesh = plsc.VectorSubcoreMesh(core_axis_name="c", subcore_axis_name="s")


def _wid(cid, sid):
    return sid * NC + cid


@functools.partial(
    pl.kernel,
    out_type=jax.ShapeDtypeStruct((NC * 8, NPAD), jnp.float32),
    mesh=_mesh,
    scratch_types=[
        pltpu.VMEM((RPT, CHUNK), jnp.int32),
        pltpu.VMEM((RPT, CHUNK), jnp.float32),
        pltpu.VMEM((PER_SC,), jnp.float32),
        pltpu.VMEM_SHARED((NPAD,), jnp.float32),
    ],
)
def _deg_kernel(dst_hbm, ew_hbm, out_hbm, dst_v, ew_v, zbuf, acc):
    cid = lax.axis_index("c")
    sid = lax.axis_index("s")
    wid = _wid(cid, sid)

    zeros16 = jnp.zeros((16,), jnp.float32)
    for g in range(PER_SC // 16):
        zbuf[pl.ds(g * 16, 16)] = zeros16
    pltpu.sync_copy(zbuf, acc.at[pl.ds(sid * PER_SC, PER_SC)])
    plsc.subcore_barrier()

    pltpu.sync_copy(dst_hbm.at[pl.ds(wid * RPT, RPT)], dst_v)
    pltpu.sync_copy(ew_hbm.at[pl.ds(wid * RPT, RPT)], ew_v)

    def body(j, carry):
        pltpu.sync_copy(ew_v.at[j], acc.at[dst_v.at[j]], add=True)
        return carry

    lax.fori_loop(0, RPT, body, 0)
    plsc.subcore_barrier()

    pltpu.sync_copy(acc.at[pl.ds(sid * PER_SC, PER_SC)],
                    out_hbm.at[cid * 8, pl.ds(sid * PER_SC, PER_SC)])


def _make_agg_kernel(D):
    epg = 16 // D
    ngroups = (CHUNK * D) // 16
    logd = D.bit_length() - 1

    @functools.partial(
        pl.kernel,
        out_type=jax.ShapeDtypeStruct((NC, NPAD, D), jnp.float32),
        mesh=_mesh,
        compiler_params=pltpu.CompilerParams(
            use_tc_tiling_on_sc=False, needs_layout_passes=False),
        scratch_types=[
            pltpu.VMEM((RPT, CHUNK), jnp.int32),
            pltpu.VMEM((RPT, CHUNK), jnp.int32),
            pltpu.VMEM((RPT, CHUNK), jnp.float32),
            pltpu.VMEM((CHUNK, D), jnp.float32),
            pltpu.VMEM((CHUNK, D), jnp.float32),
            pltpu.VMEM((CHUNK, D), jnp.float32),
            pltpu.VMEM((CHUNK, D), jnp.float32),
            pltpu.VMEM_SHARED((NPAD, D), jnp.float32),
            pltpu.SemaphoreType.DMA,
            pltpu.SemaphoreType.DMA,
            pltpu.SemaphoreType.DMA,
            pltpu.SemaphoreType.DMA,
        ],
    )
    def _agg(src_hbm, dst_hbm, ew_hbm, h_hbm, out_hbm,
             src_v, dst_v, ew_v, rows0, rows1, msg0, msg1, acc,
             sem0, sem1, ssem0, ssem1):
        cid = lax.axis_index("c")
        sid = lax.axis_index("s")
        wid = _wid(cid, sid)

        zeros16 = jnp.zeros((16,), jnp.float32)
        iota = lax.iota(jnp.int32, 16)
        erel = lax.shift_right_logical(iota, logd)
        cidx_c = lax.bitwise_and(iota, D - 1)
        eidx_c = [erel + g * epg for g in range(ngroups)]

        for g in range(ngroups):
            plsc.store_scatter(msg0, [eidx_c[g], cidx_c], zeros16)
            plsc.store_scatter(msg1, [eidx_c[g], cidx_c], zeros16)
        for r in range(PER_SC // CHUNK):
            pltpu.sync_copy(
                msg0, acc.at[pl.ds(sid * PER_SC + r * CHUNK, CHUNK)])
        plsc.subcore_barrier()

        pltpu.sync_copy(src_hbm.at[pl.ds(wid * RPT, RPT)], src_v)
        pltpu.sync_copy(dst_hbm.at[pl.ds(wid * RPT, RPT)], dst_v)
        pltpu.sync_copy(ew_hbm.at[pl.ds(wid * RPT, RPT)], ew_v)

        rows = (rows0, rows1)
        sems = (sem0, sem1)
        msgs = (msg0, msg1)
        ssems = (ssem0, ssem1)
        NBUF = 2

        for b in range(NBUF):
            pltpu.async_copy(h_hbm.at[src_v.at[b]], rows[b], sems[b])
            pltpu.async_copy(msgs[b], acc.at[dst_v.at[b]], ssems[b], add=True)

        def _process(j, rv, mv):
            j16 = jnp.full((16,), j, jnp.int32)
            for g in range(ngroups):
                ewb = plsc.load_gather(ew_v, [j16, eidx_c[g]])
                if D == 16:
                    mv[g] = ewb * rv[g]
                else:
                    vals = plsc.load_gather(rv, [eidx_c[g], cidx_c])
                    plsc.store_scatter(mv, [eidx_c[g], cidx_c], ewb * vals)

        def body(i, carry):
            j0 = NBUF * i
            for b in range(NBUF):
                j = j0 + b
                pltpu.make_async_copy(
                    h_hbm.at[src_v.at[j]], rows[b], sems[b]).wait()
                pltpu.make_async_copy(
                    msgs[b], acc.at[dst_v.at[j]], ssems[b]).wait()
                _process(j, rows[b], msgs[b])
                pltpu.async_copy(
                    msgs[b], acc.at[dst_v.at[j]], ssems[b], add=True)
                jn = jnp.minimum(j + NBUF, RPT - 1)

                @pl.when(j + NBUF < RPT)
                def _():
                    pltpu.async_copy(h_hbm.at[src_v.at[jn]], rows[b], sems[b])

            return carry

        lax.fori_loop(0, RPT // NBUF, body, 0)
        for b in range(NBUF):
            pltpu.make_async_copy(
                msgs[b], acc.at[dst_v.at[b]], ssems[b]).wait()
        plsc.subcore_barrier()

        pltpu.sync_copy(acc.at[pl.ds(sid * PER_SC, PER_SC)],
                        out_hbm.at[cid, pl.ds(sid * PER_SC, PER_SC)])

    return _agg


_agg16 = _make_agg_kernel(16)


_BLK = 1000
_GRID = N // _BLK


def _tc_a_body(degp_ref, x_ref, w1_ref, dinv_ref, h1s_ref):
    deg = 1.0 + degp_ref[0] + degp_ref[1]
    dinv = lax.rsqrt(deg)
    dinv_ref[...] = dinv
    h = jnp.dot(x_ref[...], w1_ref[...], preferred_element_type=jnp.float32)
    h1s_ref[...] = h * dinv


def _tc_b_body(s1p_ref, h1s_ref, dinv_ref, b1_ref, w2p_ref, h2s_ref):
    dinv = dinv_ref[...]
    pre = dinv * (s1p_ref[0] + s1p_ref[1] + h1s_ref[...]) + b1_ref[...]
    z = jnp.maximum(pre, 0.0)
    h2 = jnp.dot(z, w2p_ref[...], preferred_element_type=jnp.float32)
    h2s_ref[...] = h2 * dinv


def _tc_c_body(s2p_ref, h2s_ref, dinv_ref, b2_ref, out_ref):
    o2 = dinv_ref[...] * (s2p_ref[0] + s2p_ref[1] + h2s_ref[...]) + b2_ref[...]
    m = jnp.max(o2, axis=1, keepdims=True)
    lse = m + jnp.log(jnp.sum(jnp.exp(o2 - m), axis=1, keepdims=True))
    out_ref[...] = o2 - lse


def kernel(x, edge_index, edge_weight, W1, b1, W2, b2):
    f32 = jnp.float32
    src = edge_index[0].astype(jnp.int32)
    dst = edge_index[1].astype(jnp.int32)
    ew = edge_weight.astype(f32)

    pad = EPAD - E
    pad_idx = jnp.arange(pad, dtype=jnp.int32) % N
    src2d = jnp.concatenate([src, pad_idx]).reshape(ROWS, CHUNK)
    dst2d = jnp.concatenate([dst, pad_idx]).reshape(ROWS, CHUNK)
    ew2d = jnp.concatenate([ew, jnp.zeros((pad,), f32)]).reshape(ROWS, CHUNK)

    w2p = jnp.concatenate([W2, jnp.zeros((16, 14), f32)], axis=1)

    degp = _deg_kernel(dst2d, ew2d)[::8]
    degp3 = degp[:, :N, None]

    dinv, h1s = pl.pallas_call(
        _tc_a_body,
        grid=(_GRID,),
        in_specs=[
            pl.BlockSpec((NC, _BLK, 1), lambda i: (0, i, 0)),
            pl.BlockSpec((_BLK, 128), lambda i: (i, 0)),
            pl.BlockSpec((128, 16), lambda i: (0, 0)),
        ],
        out_specs=[
            pl.BlockSpec((_BLK, 1), lambda i: (i, 0)),
            pl.BlockSpec((_BLK, 16), lambda i: (i, 0)),
        ],
        out_shape=[
            jax.ShapeDtypeStruct((N, 1), f32),
            jax.ShapeDtypeStruct((N, 16), f32),
        ],
    )(degp3, x, W1)

    s1p = _agg16(src2d, dst2d, ew2d, h1s)
    s1p = s1p[:, :N, :]

    h2s = pl.pallas_call(
        _tc_b_body,
        grid=(_GRID,),
        in_specs=[
            pl.BlockSpec((NC, _BLK, 16), lambda i: (0, i, 0)),
            pl.BlockSpec((_BLK, 16), lambda i: (i, 0)),
            pl.BlockSpec((_BLK, 1), lambda i: (i, 0)),
            pl.BlockSpec((16,), lambda i: (0,)),
            pl.BlockSpec((16, 16), lambda i: (0, 0)),
        ],
        out_specs=pl.BlockSpec((_BLK, 16), lambda i: (i, 0)),
        out_shape=jax.ShapeDtypeStruct((N, 16), f32),
    )(s1p, h1s, dinv, b1, w2p)

    s2p = _agg16(src2d, dst2d, ew2d, h2s)
    s2p = s2p[:, :N, :2]
    h2s2 = h2s[:, :2]

    out = pl.pallas_call(
        _tc_c_body,
        grid=(_GRID,),
        in_specs=[
            pl.BlockSpec((NC, _BLK, 2), lambda i: (0, i, 0)),
            pl.BlockSpec((_BLK, 2), lambda i: (i, 0)),
            pl.BlockSpec((_BLK, 1), lambda i: (i, 0)),
            pl.BlockSpec((2,), lambda i: (0,)),
        ],
        out_specs=pl.BlockSpec((_BLK, 2), lambda i: (i, 0)),
        out_shape=jax.ShapeDtypeStruct((N, 2), f32),
    )(s2p, h2s2, dinv, b2)

    return out
